# baseline (speedup 1.0000x reference)
import functools
import math

import jax
import jax.numpy as jnp
from jax import lax
from jax.experimental import pallas as pl
from jax.experimental.pallas import tpu as pltpu
from jax.experimental.pallas import tpu_sc as plsc

DIM = 64
MAX_LEN = 5000

_NC = 2
_NS = 16
_NW = _NC * _NS

_IDX_W = 128


def _sc_gather(table, idx, n_rows):
    per_w = n_rows // _NW
    chunk = 640
    n_chunks = per_w // chunk
    n_streams = chunk // _IDX_W

    mesh = plsc.VectorSubcoreMesh(
        core_axis_name="c", subcore_axis_name="s",
        num_cores=_NC, num_subcores=_NS)

    @functools.partial(
        pl.kernel,
        mesh=mesh,
        out_type=jax.ShapeDtypeStruct((n_rows, DIM), jnp.float32),
        scratch_types=[
            pltpu.VMEM((chunk,), jnp.int32),
            pltpu.VMEM((chunk, DIM), jnp.float32),
            pltpu.SemaphoreType.DMA,
        ],
        compiler_params=pltpu.CompilerParams(use_tc_tiling_on_sc=False),
    )
    def k(table_hbm, idx_hbm, out_hbm, idx_v, rows_v, sem):
        wid = lax.axis_index("s") * _NC + lax.axis_index("c")
        wbase = wid * per_w

        def body(c, _):
            base = wbase + c * chunk
            pltpu.sync_copy(idx_hbm.at[pl.ds(base, chunk)], idx_v)
            descs = [
                pltpu.async_copy(
                    table_hbm.at[idx_v.at[pl.ds(j * _IDX_W, _IDX_W)]],
                    rows_v.at[pl.ds(j * _IDX_W, _IDX_W)],
                    sem)
                for j in range(n_streams)
            ]
            for d in descs:
                d.wait()
            pltpu.sync_copy(rows_v, out_hbm.at[pl.ds(base, chunk)])
            return ()

        lax.fori_loop(0, n_chunks, body, (), unroll=False)

    return k(table, idx)


def _ln_body(emb_ref, pe_ref, g_ref, b_ref, out_ref):
    e = emb_ref[...] + pe_ref[...]
    mu = jnp.mean(e, axis=-1, keepdims=True)
    var = jnp.mean(jnp.square(e - mu), axis=-1, keepdims=True)
    out_ref[...] = (e - mu) * lax.rsqrt(var + 1e-5) * g_ref[...] + b_ref[...]


def _tc_ln(emb, pe, gamma, beta):
    b, l, d = emb.shape
    bb = 16
    return pl.pallas_call(
        _ln_body,
        grid=(b // bb,),
        in_specs=[
            pl.BlockSpec((bb, l, d), lambda i: (i, 0, 0)),
            pl.BlockSpec((1, l, d), lambda i: (0, 0, 0)),
            pl.BlockSpec((1, 1, d), lambda i: (0, 0, 0)),
            pl.BlockSpec((1, 1, d), lambda i: (0, 0, 0)),
        ],
        out_specs=pl.BlockSpec((bb, l, d), lambda i: (i, 0, 0)),
        out_shape=jax.ShapeDtypeStruct((b, l, d), jnp.float32),
    )(emb, pe, gamma, beta)


def _pe_table(length, d):
    position = jnp.arange(length, dtype=jnp.float32)[:, None]
    div_term = jnp.exp(
        jnp.arange(0, d, 2, dtype=jnp.float32) * (-math.log(10000.0) / d))
    ang = position * div_term
    pe = jnp.zeros((length, d), dtype=jnp.float32)
    pe = pe.at[:, 0::2].set(jnp.sin(ang))
    pe = pe.at[:, 1::2].set(jnp.cos(ang))
    return pe[None]


def kernel(x, word_embeddings_weight, ln_gamma, ln_beta):
    b, l = x.shape
    n = b * l
    gathered = _sc_gather(word_embeddings_weight, x.reshape(n), n)
    pe = _pe_table(l, DIM)
    g = ln_gamma.reshape(1, 1, DIM)
    be = ln_beta.reshape(1, 1, DIM)
    return _tc_ln(gathered.reshape(b, l, DIM), pe, g, be)


# speedup vs baseline: 1.1281x; 1.1281x over previous
import functools
import math

import jax
import jax.numpy as jnp
from jax import lax
from jax.experimental import pallas as pl
from jax.experimental.pallas import tpu as pltpu
from jax.experimental.pallas import tpu_sc as plsc

DIM = 64
MAX_LEN = 5000

_NC = 2
_NS = 16
_NW = _NC * _NS

_IDX_W = 128


def _sc_gather(table, idx, n_rows):
    per_w = n_rows // _NW
    chunk = 320
    n_chunks = per_w // chunk
    n_streams = chunk // _IDX_W

    mesh = plsc.VectorSubcoreMesh(
        core_axis_name="c", subcore_axis_name="s",
        num_cores=_NC, num_subcores=_NS)

    @functools.partial(
        pl.kernel,
        mesh=mesh,
        out_type=jax.ShapeDtypeStruct((n_rows, 2 * DIM), jnp.float32),
        scratch_types=[
            pltpu.VMEM((chunk,), jnp.int32),
            pltpu.VMEM((chunk, 2 * DIM), jnp.float32),
            pltpu.SemaphoreType.DMA,
        ],
        compiler_params=pltpu.CompilerParams(use_tc_tiling_on_sc=False),
    )
    def k(table_hbm, idx_hbm, out_hbm, idx_v, rows_v, sem):
        wid = lax.axis_index("s") * _NC + lax.axis_index("c")
        wbase = wid * per_w

        def body(c, _):
            base = wbase + c * chunk
            pltpu.sync_copy(idx_hbm.at[pl.ds(base, chunk)], idx_v)
            descs = [
                pltpu.async_copy(
                    table_hbm.at[idx_v.at[pl.ds(j * _IDX_W, _IDX_W)]],
                    rows_v.at[pl.ds(j * _IDX_W, _IDX_W)],
                    sem)
                for j in range(n_streams)
            ]
            for d in descs:
                d.wait()
            pltpu.sync_copy(rows_v, out_hbm.at[pl.ds(base, chunk)])
            return ()

        lax.fori_loop(0, n_chunks, body, (), unroll=False)

    return k(table, idx)


def _ln_body(emb_ref, pe_ref, g_ref, b_ref, out_ref):
    e = emb_ref[..., :DIM] + pe_ref[...]
    mu = jnp.mean(e, axis=-1, keepdims=True)
    var = jnp.mean(jnp.square(e - mu), axis=-1, keepdims=True)
    out_ref[...] = (e - mu) * lax.rsqrt(var + 1e-5) * g_ref[...] + b_ref[...]


def _tc_ln(emb, pe, gamma, beta):
    b, l, d2 = emb.shape
    d = DIM
    bb = 16
    return pl.pallas_call(
        _ln_body,
        grid=(b // bb,),
        in_specs=[
            pl.BlockSpec((bb, l, d2), lambda i: (i, 0, 0)),
            pl.BlockSpec((1, l, d), lambda i: (0, 0, 0)),
            pl.BlockSpec((1, 1, d), lambda i: (0, 0, 0)),
            pl.BlockSpec((1, 1, d), lambda i: (0, 0, 0)),
        ],
        out_specs=pl.BlockSpec((bb, l, d), lambda i: (i, 0, 0)),
        out_shape=jax.ShapeDtypeStruct((b, l, d), jnp.float32),
    )(emb, pe, gamma, beta)


def _pe_table(length, d):
    position = jnp.arange(length, dtype=jnp.float32)[:, None]
    div_term = jnp.exp(
        jnp.arange(0, d, 2, dtype=jnp.float32) * (-math.log(10000.0) / d))
    ang = position * div_term
    pe = jnp.zeros((length, d), dtype=jnp.float32)
    pe = pe.at[:, 0::2].set(jnp.sin(ang))
    pe = pe.at[:, 1::2].set(jnp.cos(ang))
    return pe[None]


def kernel(x, word_embeddings_weight, ln_gamma, ln_beta):
    b, l = x.shape
    n = b * l
    table128 = jnp.pad(word_embeddings_weight, ((0, 0), (0, DIM)))
    gathered = _sc_gather(table128, x.reshape(n), n)
    pe = _pe_table(l, DIM)
    g = ln_gamma.reshape(1, 1, DIM)
    be = ln_beta.reshape(1, 1, DIM)
    return _tc_ln(gathered.reshape(b, l, 2 * DIM), pe, g, be)
